# baseline (speedup 1.0000x reference)
import dataclasses
import functools

import jax
import jax.numpy as jnp
from jax import lax
from jax.experimental import pallas as pl
from jax.experimental.pallas import tpu as pltpu
from jax.experimental.pallas import tpu_sc as plsc

B = 16384
F = 64
NC = 2
NS = 16
NW = NC * NS
BPW = B // NW
NCHUNK = BPW // 128


def _sc_predict(uids2d, iids2d, user_factors, item_factors, ub_flat, ib_flat, gb16):
    mesh = plsc.VectorSubcoreMesh(core_axis_name="c", subcore_axis_name="s")
    cp = pltpu.CompilerParams()
    if "needs_layout_passes" in pltpu.CompilerParams.__dataclass_fields__:
        cp = dataclasses.replace(cp, needs_layout_passes=False)
    if "use_tc_tiling_on_sc" in pltpu.CompilerParams.__dataclass_fields__:
        cp = dataclasses.replace(cp, use_tc_tiling_on_sc=False)

    @functools.partial(
        pl.kernel,
        mesh=mesh,
        compiler_params=cp,
        out_type=jax.ShapeDtypeStruct((B,), jnp.float32),
        scratch_types=[
            pltpu.VMEM((NCHUNK, 128), jnp.int32),
            pltpu.VMEM((NCHUNK, 128), jnp.int32),
            pltpu.VMEM((BPW, F), jnp.float32),
            pltpu.VMEM((BPW, F), jnp.float32),
            pltpu.VMEM((BPW,), jnp.float32),
            pltpu.VMEM((BPW,), jnp.float32),
            pltpu.VMEM((BPW + 16,), jnp.float32),
            pltpu.VMEM((BPW,), jnp.float32),
            pltpu.VMEM((16,), jnp.float32),
            pltpu.SemaphoreType.DMA,
        ],
    )
    def body(uids_hbm, iids_hbm, uf_hbm, if_hbm, ubias_hbm, ibias_hbm, gb_hbm,
             out_hbm, idx_u, idx_i, u_rows, i_rows, ub_v, ib_v, dots, out_v,
             gb_v, sem):
        wid = lax.axis_index("s") * NC + lax.axis_index("c")
        base = wid * BPW

        pltpu.sync_copy(uids_hbm.at[pl.ds(wid * NCHUNK, NCHUNK)], idx_u)
        pltpu.sync_copy(iids_hbm.at[pl.ds(wid * NCHUNK, NCHUNK)], idx_i)
        pltpu.sync_copy(gb_hbm, gb_v)

        handles = []
        for j in range(NCHUNK):
            dst = pl.ds(j * 128, 128)
            handles.append(
                pltpu.async_copy(uf_hbm.at[idx_u.at[j]], u_rows.at[dst], sem))
            handles.append(
                pltpu.async_copy(if_hbm.at[idx_i.at[j]], i_rows.at[dst], sem))
            handles.append(
                pltpu.async_copy(ubias_hbm.at[idx_u.at[j]], ub_v.at[dst], sem))
            handles.append(
                pltpu.async_copy(ibias_hbm.at[idx_i.at[j]], ib_v.at[dst], sem))
        for h in handles:
            h.wait()

        lane = lax.iota(jnp.int32, 16)
        last_lane = lane == 15

        @pl.loop(0, BPW)
        def _(r):
            p = u_rows[r, pl.ds(0, 16)] * i_rows[r, pl.ds(0, 16)]
            for c in range(1, F // 16):
                p += u_rows[r, pl.ds(c * 16, 16)] * i_rows[r, pl.ds(c * 16, 16)]
            cs = plsc.cumsum(p)
            plsc.store_compressed(dots.at[pl.ds(r, 16)], cs, mask=last_lane)

        gb_vec = gb_v[...]

        @pl.loop(0, BPW, step=16)
        def _(c):
            d = dots[pl.ds(c, 16)]
            pred = d + ub_v[pl.ds(c, 16)] + ib_v[pl.ds(c, 16)] + gb_vec
            pred = jnp.minimum(jnp.maximum(pred, 1.0), 5.0)
            out_v[pl.ds(c, 16)] = pred

        pltpu.sync_copy(out_v, out_hbm.at[pl.ds(base, BPW)])

    return body(uids2d, iids2d, user_factors, item_factors, ub_flat, ib_flat,
                gb16)


def kernel(user_ids, item_ids, user_factors, item_factors, user_biases,
           item_biases, global_bias):
    uids2d = user_ids.reshape(NW * NCHUNK, 128)
    iids2d = item_ids.reshape(NW * NCHUNK, 128)
    ub_flat = user_biases.reshape(-1)
    ib_flat = item_biases.reshape(-1)
    gb16 = jnp.broadcast_to(global_bias.astype(jnp.float32), (16,))
    return _sc_predict(uids2d, iids2d, user_factors, item_factors, ub_flat,
                       ib_flat, gb16)


# speedup vs baseline: 1.0011x; 1.0011x over previous
import dataclasses
import functools

import jax
import jax.numpy as jnp
from jax import lax
from jax.experimental import pallas as pl
from jax.experimental.pallas import tpu as pltpu
from jax.experimental.pallas import tpu_sc as plsc

B = 16384
F = 64
NC = 2
NS = 16
NW = NC * NS
BPW = B // NW
NCHUNK = BPW // 128


def _sc_predict(uids2d, iids2d, user_factors, item_factors, ub_flat, ib_flat, gb16):
    mesh = plsc.VectorSubcoreMesh(core_axis_name="c", subcore_axis_name="s")
    cp = pltpu.CompilerParams()
    if "needs_layout_passes" in pltpu.CompilerParams.__dataclass_fields__:
        cp = dataclasses.replace(cp, needs_layout_passes=False)
    if "use_tc_tiling_on_sc" in pltpu.CompilerParams.__dataclass_fields__:
        cp = dataclasses.replace(cp, use_tc_tiling_on_sc=False)

    @functools.partial(
        pl.kernel,
        mesh=mesh,
        compiler_params=cp,
        out_type=jax.ShapeDtypeStruct((B,), jnp.float32),
        scratch_types=[
            pltpu.VMEM((NCHUNK, 128), jnp.int32),
            pltpu.VMEM((NCHUNK, 128), jnp.int32),
            pltpu.VMEM((BPW, F), jnp.float32),
            pltpu.VMEM((BPW, F), jnp.float32),
            pltpu.VMEM((NCHUNK, 128), jnp.int32),
            pltpu.VMEM((NCHUNK, 128), jnp.int32),
            pltpu.VMEM((BPW, 16), jnp.float32),
            pltpu.VMEM((BPW, 16), jnp.float32),
            pltpu.VMEM((BPW + 16,), jnp.float32),
            pltpu.VMEM((BPW,), jnp.float32),
            pltpu.VMEM((16,), jnp.float32),
            pltpu.SemaphoreType.DMA,
        ],
    )
    def body(uids_hbm, iids_hbm, uf_hbm, if_hbm, ubias_hbm, ibias_hbm, gb_hbm,
             out_hbm, idx_u, idx_i, u_rows, i_rows, idx_su, idx_si, ub_g, ib_g,
             dots, out_v, gb_v, sem):
        wid = lax.axis_index("s") * NC + lax.axis_index("c")
        base = wid * BPW

        pltpu.sync_copy(uids_hbm.at[pl.ds(wid * NCHUNK, NCHUNK)], idx_u)
        pltpu.sync_copy(iids_hbm.at[pl.ds(wid * NCHUNK, NCHUNK)], idx_i)
        pltpu.sync_copy(gb_hbm, gb_v)

        for j in range(NCHUNK):
            for k in range(8):
                s = pl.ds(k * 16, 16)
                idx_su[j, s] = lax.shift_right_logical(idx_u[j, s], 4)
                idx_si[j, s] = lax.shift_right_logical(idx_i[j, s], 4)

        handles = []
        for j in range(NCHUNK):
            dst = pl.ds(j * 128, 128)
            handles.append(
                pltpu.async_copy(uf_hbm.at[idx_u.at[j]], u_rows.at[dst], sem))
            handles.append(
                pltpu.async_copy(if_hbm.at[idx_i.at[j]], i_rows.at[dst], sem))
            handles.append(
                pltpu.async_copy(ubias_hbm.at[idx_su.at[j]], ub_g.at[dst], sem))
            handles.append(
                pltpu.async_copy(ibias_hbm.at[idx_si.at[j]], ib_g.at[dst], sem))
        for h in handles:
            h.wait()

        lane = lax.iota(jnp.int32, 16)
        last_lane = lane == 15

        @pl.loop(0, BPW)
        def _(r):
            p = u_rows[r, pl.ds(0, 16)] * i_rows[r, pl.ds(0, 16)]
            for c in range(1, F // 16):
                p += u_rows[r, pl.ds(c * 16, 16)] * i_rows[r, pl.ds(c * 16, 16)]
            cs = plsc.cumsum(p)
            plsc.store_compressed(dots.at[pl.ds(r, 16)], cs, mask=last_lane)

        gb_vec = gb_v[...]

        for j in range(NCHUNK):
            @pl.loop(0, 128, step=16)
            def _(off, j=j):
                c = j * 128 + off
                d = dots[pl.ds(c, 16)]
                row = lane + c
                mod_u = idx_u[j, pl.ds(off, 16)] & 15
                mod_i = idx_i[j, pl.ds(off, 16)] & 15
                ub = plsc.load_gather(ub_g, [row, mod_u])
                ib = plsc.load_gather(ib_g, [row, mod_i])
                pred = d + ub + ib + gb_vec
                pred = jnp.minimum(jnp.maximum(pred, 1.0), 5.0)
                out_v[pl.ds(c, 16)] = pred

        pltpu.sync_copy(out_v, out_hbm.at[pl.ds(base, BPW)])

    return body(uids2d, iids2d, user_factors, item_factors, ub_flat, ib_flat,
                gb16)


def kernel(user_ids, item_ids, user_factors, item_factors, user_biases,
           item_biases, global_bias):
    uids2d = user_ids.reshape(NW * NCHUNK, 128)
    iids2d = item_ids.reshape(NW * NCHUNK, 128)
    ub16 = user_biases.reshape(-1, 16)
    ib16 = item_biases.reshape(-1, 16)
    gb16 = jnp.broadcast_to(global_bias.astype(jnp.float32), (16,))
    return _sc_predict(uids2d, iids2d, user_factors, item_factors, ub16,
                       ib16, gb16)
